# Initial kernel scaffold; baseline (speedup 1.0000x reference)
import jax
import jax.numpy as jnp
from jax.experimental import pallas as pl


def kernel(X, edge_index, W_in, b_in, W1, b1, g1, be1, W2, b2, g2, be2, Wo1, bo1, Wo2, bo2):
    raise NotImplementedError("write your pallas kernel here")


# speedup vs baseline: 18.1564x; 18.1564x over previous
import functools

import jax
import jax.numpy as jnp
from jax import lax
from jax.experimental import pallas as pl
from jax.experimental.pallas import tpu as pltpu
from jax.experimental.pallas import tpu_sc as plsc

NC, NS = 2, 16
NW = NC * NS
CH = 128


def _sc_degree(dst3, n_pad, nch):
    mesh = plsc.VectorSubcoreMesh(core_axis_name="c", subcore_axis_name="s")

    @functools.partial(
        pl.kernel,
        out_type=jax.ShapeDtypeStruct((NC, n_pad), jnp.float32),
        mesh=mesh,
        scratch_types=[
            pltpu.VMEM((nch, CH), jnp.int32),
            pltpu.VMEM((CH,), jnp.float32),
            pltpu.VMEM((n_pad // NS,), jnp.float32),
            pltpu.VMEM_SHARED((n_pad,), jnp.float32),
            pltpu.SemaphoreType.DMA,
        ],
    )
    def deg_kernel(dst_hbm, out_hbm, didx, ones_v, zslab, acc_sh, sem):
        c = lax.axis_index("c")
        s = lax.axis_index("s")
        wid = s * NC + c
        pltpu.sync_copy(dst_hbm.at[wid], didx)

        def init(i, _):
            ones_v[pl.ds(i * 16, 16)] = jnp.ones((16,), jnp.float32)
            return 0
        lax.fori_loop(0, CH // 16, init, 0)

        slab = n_pad // NS

        def zinit(i, _):
            zslab[pl.ds(i * 16, 16)] = jnp.zeros((16,), jnp.float32)
            return 0
        lax.fori_loop(0, slab // 16, zinit, 0)
        pltpu.sync_copy(zslab, acc_sh.at[pl.ds(s * slab, slab)])
        plsc.subcore_barrier()

        cps = [pltpu.async_copy(ones_v, acc_sh.at[didx.at[j]], sem, add=True)
               for j in range(nch)]
        for cp in cps:
            cp.wait()
        plsc.subcore_barrier()
        pltpu.sync_copy(acc_sh.at[pl.ds(s * slab, slab)],
                        out_hbm.at[c].at[pl.ds(s * slab, slab)])

    return deg_kernel(dst3)


def _sc_msgpass(xs_split, src3, dst3, n_pad, nch, dh):
    mesh = plsc.VectorSubcoreMesh(core_axis_name="c", subcore_axis_name="s")
    rps = n_pad // (NS * CH)

    @functools.partial(
        pl.kernel,
        out_type=jax.ShapeDtypeStruct((NC, n_pad, dh), jnp.float32),
        mesh=mesh,
        scratch_types=[
            pltpu.VMEM((nch, CH), jnp.int32),
            pltpu.VMEM((nch, CH), jnp.int32),
            pltpu.VMEM((CH, dh), jnp.float32),
            pltpu.VMEM((CH, dh), jnp.float32),
            pltpu.VMEM_SHARED((n_pad, dh), jnp.float32),
            pltpu.SemaphoreType.DMA,
            pltpu.SemaphoreType.DMA,
        ],
        compiler_params=pltpu.CompilerParams(use_tc_tiling_on_sc=False),
    )
    def mp_kernel(xs_hbm, src_hbm, dst_hbm, out_hbm,
                  sidx, didx, rows0, rows1, acc_sh, sem0, sem1):
        c = lax.axis_index("c")
        s = lax.axis_index("s")
        pltpu.sync_copy(src_hbm.at[s], sidx)
        pltpu.sync_copy(dst_hbm.at[s], didx)

        def zr(r, _):
            for cc in range(dh // 16):
                rows0[r, pl.ds(cc * 16, 16)] = jnp.zeros((16,), jnp.float32)
            return 0
        lax.fori_loop(0, CH, zr, 0)
        for k in range(rps):
            pltpu.sync_copy(rows0, acc_sh.at[pl.ds((s * rps + k) * CH, CH)])
        plsc.subcore_barrier()

        tbl = xs_hbm.at[c]
        bufs = (rows0, rows1)
        sems = (sem0, sem1)
        cps = [None, None]
        cps[0] = pltpu.async_copy(tbl.at[sidx.at[0]], bufs[0], sems[0])
        for j in range(nch):
            if j + 1 < nch:
                nb = (j + 1) % 2
                cps[nb] = pltpu.async_copy(tbl.at[sidx.at[j + 1]], bufs[nb], sems[nb])
            cps[j % 2].wait()
            pltpu.sync_copy(bufs[j % 2], acc_sh.at[didx.at[j]], add=True)
        plsc.subcore_barrier()
        for k in range(rps):
            off = (s * rps + k) * CH
            pltpu.sync_copy(acc_sh.at[pl.ds(off, CH)],
                            out_hbm.at[c].at[pl.ds(off, CH)])

    return mp_kernel(xs_split, src3, dst3)


def _tc_pre(xl, d3, W_in, b_in, W1, n):
    h1 = W1.shape[1]
    dh = h1 // NC

    def body(xl_ref, d_ref, wi_ref, bi_ref, w1_ref, dinv_ref, xs1_ref):
        deg = d_ref[0, :n, :] + d_ref[1, :n, :] + 1.0
        dinv = lax.rsqrt(deg)
        h0 = jnp.dot(xl_ref[...], wi_ref[...],
                     preferred_element_type=jnp.float32) + bi_ref[...][None, :]
        xs1 = jnp.dot(h0, w1_ref[...],
                      preferred_element_type=jnp.float32) * dinv
        dinv_ref[...] = dinv
        for c in range(NC):
            xs1_ref[c] = xs1[:, c * dh:(c + 1) * dh]

    return pl.pallas_call(
        body,
        out_shape=(jax.ShapeDtypeStruct((n, 1), jnp.float32),
                   jax.ShapeDtypeStruct((NC, n, dh), jnp.float32)),
    )(xl, d3, W_in, b_in, W1)


def _bn(y, g, b):
    mu = jnp.mean(y, axis=0)
    var = jnp.mean((y - mu) ** 2, axis=0)
    return (y - mu) * lax.rsqrt(var + 1e-5) * g + b


def _tc_mid(a, xs1, dinv, b1, g1, be1, W2, n):
    h2 = W2.shape[1]
    dh = h2 // NC

    def body(a_ref, xs1_ref, dinv_ref, b1_ref, g1_ref, be1_ref, w2_ref, xs2_ref):
        dinv = dinv_ref[...]
        agg = jnp.concatenate(
            [a_ref[c, :n, :] + xs1_ref[c] for c in range(NC)], axis=1)
        y = agg * dinv + b1_ref[...][None, :]
        y = _bn(y, g1_ref[...][None, :], be1_ref[...][None, :])
        y = jnp.maximum(y, 0.0)
        xs2 = jnp.dot(y, w2_ref[...],
                      preferred_element_type=jnp.float32) * dinv
        for c in range(NC):
            xs2_ref[c] = xs2[:, c * dh:(c + 1) * dh]

    return pl.pallas_call(
        body,
        out_shape=jax.ShapeDtypeStruct((NC, n, dh), jnp.float32),
    )(a, xs1, dinv, b1, g1, be1, W2)


def _tc_post(a, xs2, dinv, b2, g2, be2, Wo1, bo1, Wo2, bo2, n):
    def body(a_ref, xs2_ref, dinv_ref, b2_ref, g2_ref, be2_ref,
             wo1_ref, bo1_ref, wo2_ref, bo2_ref, out_ref):
        dinv = dinv_ref[...]
        agg = jnp.concatenate(
            [a_ref[c, :n, :] + xs2_ref[c] for c in range(NC)], axis=1)
        y = agg * dinv + b2_ref[...][None, :]
        y = _bn(y, g2_ref[...][None, :], be2_ref[...][None, :])
        h = jnp.maximum(jnp.dot(y, wo1_ref[...],
                                preferred_element_type=jnp.float32)
                        + bo1_ref[...][None, :], 0.0)
        out_ref[...] = jnp.dot(h, wo2_ref[...],
                               preferred_element_type=jnp.float32) + bo2_ref[...][None, :]

    out_dim = Wo2.shape[1]
    return pl.pallas_call(
        body,
        out_shape=jax.ShapeDtypeStruct((n, out_dim), jnp.float32),
    )(a, xs2, dinv, b2, g2, be2, Wo1, bo1, Wo2, bo2)


def kernel(X, edge_index, W_in, b_in, W1, b1, g1, be1, W2, b2, g2, be2,
           Wo1, bo1, Wo2, bo2):
    n = X.shape[0]
    e = edge_index.shape[1]
    d = W1.shape[1]

    n_pad = ((n + NS * CH - 1) // (NS * CH)) * NS * CH
    e_pad = ((e + NW * CH - 1) // (NW * CH)) * NW * CH
    nch_deg = e_pad // (NW * CH)
    nch_mp = e_pad // (NS * CH)
    pad = e_pad - e

    src_p = jnp.concatenate([edge_index[0], jnp.zeros((pad,), jnp.int32)])
    dst_p = jnp.concatenate([edge_index[1], jnp.full((pad,), n, jnp.int32)])
    dst3_deg = dst_p.reshape(NW, nch_deg, CH)
    src3 = src_p.reshape(NS, nch_mp, CH)
    dst3 = dst_p.reshape(NS, nch_mp, CH)

    xl = X[:, :, -1]
    dh = d // NC

    deg = _sc_degree(dst3_deg, n_pad, nch_deg)
    d3 = deg.reshape(NC, n_pad, 1)
    dinv, xs1 = _tc_pre(xl, d3, W_in, b_in, W1, n)
    a1 = _sc_msgpass(xs1, src3, dst3, n_pad, nch_mp, dh)
    xs2 = _tc_mid(a1, xs1, dinv, b1, g1, be1, W2, n)
    a2 = _sc_msgpass(xs2, src3, dst3, n_pad, nch_mp, dh)
    return _tc_post(a2, xs2, dinv, b2, g2, be2, Wo1, bo1, Wo2, bo2, n)


# speedup vs baseline: 18.4307x; 1.0151x over previous
import functools

import jax
import jax.numpy as jnp
from jax import lax
from jax.experimental import pallas as pl
from jax.experimental.pallas import tpu as pltpu
from jax.experimental.pallas import tpu_sc as plsc

NC, NS = 2, 16
NW = NC * NS
CH = 128
NBUF = 4


def _sc_degree(dst3, n_pad, nch):
    mesh = plsc.VectorSubcoreMesh(core_axis_name="c", subcore_axis_name="s")

    @functools.partial(
        pl.kernel,
        out_type=jax.ShapeDtypeStruct((NC, n_pad), jnp.float32),
        mesh=mesh,
        scratch_types=[
            pltpu.VMEM((nch, CH), jnp.int32),
            pltpu.VMEM((CH,), jnp.float32),
            pltpu.VMEM((n_pad // NS,), jnp.float32),
            pltpu.VMEM_SHARED((n_pad,), jnp.float32),
            pltpu.SemaphoreType.DMA,
        ],
    )
    def deg_kernel(dst_hbm, out_hbm, didx, ones_v, zslab, acc_sh, sem):
        c = lax.axis_index("c")
        s = lax.axis_index("s")
        wid = s * NC + c
        pltpu.sync_copy(dst_hbm.at[wid], didx)

        def init(i, _):
            ones_v[pl.ds(i * 16, 16)] = jnp.ones((16,), jnp.float32)
            return 0
        lax.fori_loop(0, CH // 16, init, 0)

        slab = n_pad // NS

        def zinit(i, _):
            zslab[pl.ds(i * 16, 16)] = jnp.zeros((16,), jnp.float32)
            return 0
        lax.fori_loop(0, slab // 16, zinit, 0)
        pltpu.sync_copy(zslab, acc_sh.at[pl.ds(s * slab, slab)])
        plsc.subcore_barrier()

        cps = [pltpu.async_copy(ones_v, acc_sh.at[didx.at[j]], sem, add=True)
               for j in range(nch)]
        for cp in cps:
            cp.wait()
        plsc.subcore_barrier()
        pltpu.sync_copy(acc_sh.at[pl.ds(s * slab, slab)],
                        out_hbm.at[c].at[pl.ds(s * slab, slab)])

    return deg_kernel(dst3)


def _sc_msgpass(xs_split, src3, dst3, n_pad, nch, dh):
    mesh = plsc.VectorSubcoreMesh(core_axis_name="c", subcore_axis_name="s")
    rps = n_pad // (NS * CH)

    @functools.partial(
        pl.kernel,
        out_type=jax.ShapeDtypeStruct((NC, n_pad, dh), jnp.float32),
        mesh=mesh,
        scratch_types=[
            pltpu.VMEM((nch, CH), jnp.int32),
            pltpu.VMEM((nch, CH), jnp.int32),
            pltpu.VMEM((NBUF, CH, dh), jnp.float32),
            pltpu.VMEM_SHARED((n_pad, dh), jnp.float32),
        ] + [pltpu.SemaphoreType.DMA] * (2 * NBUF),
        compiler_params=pltpu.CompilerParams(use_tc_tiling_on_sc=False),
    )
    def mp_kernel(xs_hbm, src_hbm, dst_hbm, out_hbm,
                  sidx, didx, ring, acc_sh, *sems):
        gsem = sems[:NBUF]
        ssem = sems[NBUF:]
        c = lax.axis_index("c")
        s = lax.axis_index("s")
        pltpu.sync_copy(src_hbm.at[s], sidx)
        pltpu.sync_copy(dst_hbm.at[s], didx)

        buf0 = ring.at[0]

        def zr(r, _):
            for cc in range(dh // 16):
                ring[0, r, pl.ds(cc * 16, 16)] = jnp.zeros((16,), jnp.float32)
            return 0
        lax.fori_loop(0, CH, zr, 0)
        for k in range(rps):
            pltpu.sync_copy(buf0, acc_sh.at[pl.ds((s * rps + k) * CH, CH)])
        plsc.subcore_barrier()

        tbl = xs_hbm.at[c]
        gcp = [None] * NBUF
        scp = [None] * NBUF
        for k in range(min(NBUF, nch)):
            gcp[k] = pltpu.async_copy(tbl.at[sidx.at[k]], ring.at[k], gsem[k])
        for j in range(nch):
            b = j % NBUF
            gcp[b].wait()
            scp[b] = pltpu.async_copy(ring.at[b], acc_sh.at[didx.at[j]],
                                      ssem[b], add=True)
            jl = j - (NBUF - 2)
            nj = jl + NBUF
            if jl >= 0 and nj < nch:
                bl = jl % NBUF
                scp[bl].wait()
                gcp[bl] = pltpu.async_copy(tbl.at[sidx.at[nj]], ring.at[bl],
                                           gsem[bl])
        for j in range(max(nch - NBUF, 0), nch):
            scp[j % NBUF].wait()
        plsc.subcore_barrier()
        for k in range(rps):
            off = (s * rps + k) * CH
            pltpu.sync_copy(acc_sh.at[pl.ds(off, CH)],
                            out_hbm.at[c].at[pl.ds(off, CH)])

    return mp_kernel(xs_split, src3, dst3)


def _tc_pre(xl, d3, W_in, b_in, W1, n):
    h1 = W1.shape[1]
    dh = h1 // NC

    def body(xl_ref, d_ref, wi_ref, bi_ref, w1_ref, dinv_ref, xs1_ref):
        deg = d_ref[0, :n, :] + d_ref[1, :n, :] + 1.0
        dinv = lax.rsqrt(deg)
        h0 = jnp.dot(xl_ref[...], wi_ref[...],
                     preferred_element_type=jnp.float32) + bi_ref[...][None, :]
        xs1 = jnp.dot(h0, w1_ref[...],
                      preferred_element_type=jnp.float32) * dinv
        dinv_ref[...] = dinv
        for c in range(NC):
            xs1_ref[c] = xs1[:, c * dh:(c + 1) * dh]

    return pl.pallas_call(
        body,
        out_shape=(jax.ShapeDtypeStruct((n, 1), jnp.float32),
                   jax.ShapeDtypeStruct((NC, n, dh), jnp.float32)),
    )(xl, d3, W_in, b_in, W1)


def _bn(y, g, b):
    mu = jnp.mean(y, axis=0)
    var = jnp.mean((y - mu) ** 2, axis=0)
    return (y - mu) * lax.rsqrt(var + 1e-5) * g + b


def _tc_mid(a, xs1, dinv, b1, g1, be1, W2, n):
    h2 = W2.shape[1]
    dh = h2 // NC

    def body(a_ref, xs1_ref, dinv_ref, b1_ref, g1_ref, be1_ref, w2_ref, xs2_ref):
        dinv = dinv_ref[...]
        agg = jnp.concatenate(
            [a_ref[c, :n, :] + xs1_ref[c] for c in range(NC)], axis=1)
        y = agg * dinv + b1_ref[...][None, :]
        y = _bn(y, g1_ref[...][None, :], be1_ref[...][None, :])
        y = jnp.maximum(y, 0.0)
        xs2 = jnp.dot(y, w2_ref[...],
                      preferred_element_type=jnp.float32) * dinv
        for c in range(NC):
            xs2_ref[c] = xs2[:, c * dh:(c + 1) * dh]

    return pl.pallas_call(
        body,
        out_shape=jax.ShapeDtypeStruct((NC, n, dh), jnp.float32),
    )(a, xs1, dinv, b1, g1, be1, W2)


def _tc_post(a, xs2, dinv, b2, g2, be2, Wo1, bo1, Wo2, bo2, n):
    def body(a_ref, xs2_ref, dinv_ref, b2_ref, g2_ref, be2_ref,
             wo1_ref, bo1_ref, wo2_ref, bo2_ref, out_ref):
        dinv = dinv_ref[...]
        agg = jnp.concatenate(
            [a_ref[c, :n, :] + xs2_ref[c] for c in range(NC)], axis=1)
        y = agg * dinv + b2_ref[...][None, :]
        y = _bn(y, g2_ref[...][None, :], be2_ref[...][None, :])
        h = jnp.maximum(jnp.dot(y, wo1_ref[...],
                                preferred_element_type=jnp.float32)
                        + bo1_ref[...][None, :], 0.0)
        out_ref[...] = jnp.dot(h, wo2_ref[...],
                               preferred_element_type=jnp.float32) + bo2_ref[...][None, :]

    out_dim = Wo2.shape[1]
    return pl.pallas_call(
        body,
        out_shape=jax.ShapeDtypeStruct((n, out_dim), jnp.float32),
    )(a, xs2, dinv, b2, g2, be2, Wo1, bo1, Wo2, bo2)


def kernel(X, edge_index, W_in, b_in, W1, b1, g1, be1, W2, b2, g2, be2,
           Wo1, bo1, Wo2, bo2):
    n = X.shape[0]
    e = edge_index.shape[1]
    d = W1.shape[1]

    n_pad = ((n + NS * CH - 1) // (NS * CH)) * NS * CH
    e_pad = ((e + NW * CH - 1) // (NW * CH)) * NW * CH
    nch_deg = e_pad // (NW * CH)
    nch_mp = e_pad // (NS * CH)
    pad = e_pad - e

    src_p = jnp.concatenate([edge_index[0], jnp.zeros((pad,), jnp.int32)])
    dst_p = jnp.concatenate([edge_index[1], jnp.full((pad,), n, jnp.int32)])
    dst3_deg = dst_p.reshape(NW, nch_deg, CH)
    src3 = src_p.reshape(NS, nch_mp, CH)
    dst3 = dst_p.reshape(NS, nch_mp, CH)

    xl = X[:, :, -1]
    dh = d // NC

    deg = _sc_degree(dst3_deg, n_pad, nch_deg)
    d3 = deg.reshape(NC, n_pad, 1)
    dinv, xs1 = _tc_pre(xl, d3, W_in, b_in, W1, n)
    a1 = _sc_msgpass(xs1, src3, dst3, n_pad, nch_mp, dh)
    xs2 = _tc_mid(a1, xs1, dinv, b1, g1, be1, W2, n)
    a2 = _sc_msgpass(xs2, src3, dst3, n_pad, nch_mp, dh)
    return _tc_post(a2, xs2, dinv, b2, g2, be2, Wo1, bo1, Wo2, bo2, n)


# speedup vs baseline: 25.4536x; 1.3810x over previous
import functools

import jax
import jax.numpy as jnp
from jax import lax
from jax.experimental import pallas as pl
from jax.experimental.pallas import tpu as pltpu
from jax.experimental.pallas import tpu_sc as plsc

NC, NS = 2, 16
NW = NC * NS
CH = 128
NBUF = 4
NQ = 2 * NC


def _sc_degree(dst3, n_pad, nch):
    mesh = plsc.VectorSubcoreMesh(core_axis_name="c", subcore_axis_name="s")

    @functools.partial(
        pl.kernel,
        out_type=jax.ShapeDtypeStruct((NC, n_pad), jnp.float32),
        mesh=mesh,
        scratch_types=[
            pltpu.VMEM((nch, CH), jnp.int32),
            pltpu.VMEM((CH,), jnp.float32),
            pltpu.VMEM((n_pad // NS,), jnp.float32),
            pltpu.VMEM_SHARED((n_pad,), jnp.float32),
            pltpu.SemaphoreType.DMA,
        ],
    )
    def deg_kernel(dst_hbm, out_hbm, didx, ones_v, zslab, acc_sh, sem):
        c = lax.axis_index("c")
        s = lax.axis_index("s")
        wid = s * NC + c
        pltpu.sync_copy(dst_hbm.at[wid], didx)

        def init(i, _):
            ones_v[pl.ds(i * 16, 16)] = jnp.ones((16,), jnp.float32)
            return 0
        lax.fori_loop(0, CH // 16, init, 0)

        slab = n_pad // NS

        def zinit(i, _):
            zslab[pl.ds(i * 16, 16)] = jnp.zeros((16,), jnp.float32)
            return 0
        lax.fori_loop(0, slab // 16, zinit, 0)
        pltpu.sync_copy(zslab, acc_sh.at[pl.ds(s * slab, slab)])
        plsc.subcore_barrier()

        cps = [pltpu.async_copy(ones_v, acc_sh.at[didx.at[j]], sem, add=True)
               for j in range(nch)]
        for cp in cps:
            cp.wait()
        plsc.subcore_barrier()
        pltpu.sync_copy(acc_sh.at[pl.ds(s * slab, slab)],
                        out_hbm.at[c].at[pl.ds(s * slab, slab)])

    return deg_kernel(dst3)


def _sc_msgpass(xs, src3, dst3, n_pad, nch, n, d):
    mesh = plsc.VectorSubcoreMesh(core_axis_name="c", subcore_axis_name="s")
    dq = d // NQ
    rps = n_pad // (NS * CH)
    rows_st = n // NS

    @functools.partial(
        pl.kernel,
        out_type=jax.ShapeDtypeStruct((n_pad, d), jnp.float32),
        mesh=mesh,
        scratch_types=[
            pltpu.VMEM((nch, CH), jnp.int32),
            pltpu.VMEM((nch, CH), jnp.int32),
            pltpu.VMEM((NBUF, CH, dq), jnp.float32),
            pltpu.VMEM_SHARED((n, dq), jnp.float32),
            pltpu.VMEM_SHARED((n_pad, dq), jnp.float32),
        ] + [pltpu.SemaphoreType.DMA] * (2 * NBUF),
        compiler_params=pltpu.CompilerParams(use_tc_tiling_on_sc=False),
    )
    def mp_kernel(xs_hbm, src_hbm, dst_hbm, out_hbm,
                  sidx, didx, ring, tbl_sh, acc_sh, *sems):
        gsem = sems[:NBUF]
        ssem = sems[NBUF:]
        c = lax.axis_index("c")
        s = lax.axis_index("s")
        pltpu.sync_copy(src_hbm.at[s], sidx)
        pltpu.sync_copy(dst_hbm.at[s], didx)

        for i in range(2):
            qcol = (2 * c + i) * dq
            pltpu.sync_copy(
                xs_hbm.at[pl.ds(s * rows_st, rows_st), pl.ds(qcol, dq)],
                tbl_sh.at[pl.ds(s * rows_st, rows_st)])

            def zr(r, _):
                for cc in range(dq // 16):
                    ring[0, r, pl.ds(cc * 16, 16)] = jnp.zeros((16,), jnp.float32)
                return 0
            lax.fori_loop(0, CH, zr, 0)
            for k in range(rps):
                pltpu.sync_copy(ring.at[0], acc_sh.at[pl.ds((s * rps + k) * CH, CH)])
            plsc.subcore_barrier()

            gcp = [None] * NBUF
            scp = [None] * NBUF
            for k in range(min(NBUF, nch)):
                gcp[k] = pltpu.async_copy(tbl_sh.at[sidx.at[k]], ring.at[k], gsem[k])
            for j in range(nch):
                b = j % NBUF
                gcp[b].wait()
                scp[b] = pltpu.async_copy(ring.at[b], acc_sh.at[didx.at[j]],
                                          ssem[b], add=True)
                jl = j - (NBUF - 2)
                nj = jl + NBUF
                if jl >= 0 and nj < nch:
                    bl = jl % NBUF
                    scp[bl].wait()
                    gcp[bl] = pltpu.async_copy(tbl_sh.at[sidx.at[nj]], ring.at[bl],
                                               gsem[bl])
            for j in range(max(nch - NBUF, 0), nch):
                scp[j % NBUF].wait()
            plsc.subcore_barrier()
            for k in range(rps):
                off = (s * rps + k) * CH
                pltpu.sync_copy(acc_sh.at[pl.ds(off, CH)],
                                out_hbm.at[pl.ds(off, CH), pl.ds(qcol, dq)])

    return mp_kernel(xs, src3, dst3)


def _tc_pre(xl, d3, W_in, b_in, W1, n):
    h1 = W1.shape[1]

    def body(xl_ref, d_ref, wi_ref, bi_ref, w1_ref, dinv_ref, xs1_ref):
        deg = d_ref[0, :n, :] + d_ref[1, :n, :] + 1.0
        dinv = lax.rsqrt(deg)
        h0 = jnp.dot(xl_ref[...], wi_ref[...],
                     preferred_element_type=jnp.float32) + bi_ref[...][None, :]
        xs1 = jnp.dot(h0, w1_ref[...],
                      preferred_element_type=jnp.float32) * dinv
        dinv_ref[...] = dinv
        xs1_ref[...] = xs1

    return pl.pallas_call(
        body,
        out_shape=(jax.ShapeDtypeStruct((n, 1), jnp.float32),
                   jax.ShapeDtypeStruct((n, h1), jnp.float32)),
        compiler_params=pltpu.CompilerParams(
            vmem_limit_bytes=120 * 1024 * 1024),
    )(xl, d3, W_in, b_in, W1)


def _bn(y, g, b):
    mu = jnp.mean(y, axis=0)
    var = jnp.mean((y - mu) ** 2, axis=0)
    return (y - mu) * lax.rsqrt(var + 1e-5) * g + b


def _tc_mid(a, xs1, dinv, b1, g1, be1, W2, n):
    h2 = W2.shape[1]

    def body(a_ref, xs1_ref, dinv_ref, b1_ref, g1_ref, be1_ref, w2_ref, xs2_ref):
        dinv = dinv_ref[...]
        agg = a_ref[:n, :] + xs1_ref[...]
        y = agg * dinv + b1_ref[...][None, :]
        y = _bn(y, g1_ref[...][None, :], be1_ref[...][None, :])
        y = jnp.maximum(y, 0.0)
        xs2_ref[...] = jnp.dot(y, w2_ref[...],
                               preferred_element_type=jnp.float32) * dinv

    return pl.pallas_call(
        body,
        out_shape=jax.ShapeDtypeStruct((n, h2), jnp.float32),
        compiler_params=pltpu.CompilerParams(
            vmem_limit_bytes=120 * 1024 * 1024),
    )(a, xs1, dinv, b1, g1, be1, W2)


def _tc_post(a, xs2, dinv, b2, g2, be2, Wo1, bo1, Wo2, bo2, n):
    def body(a_ref, xs2_ref, dinv_ref, b2_ref, g2_ref, be2_ref,
             wo1_ref, bo1_ref, wo2_ref, bo2_ref, out_ref):
        dinv = dinv_ref[...]
        agg = a_ref[:n, :] + xs2_ref[...]
        y = agg * dinv + b2_ref[...][None, :]
        y = _bn(y, g2_ref[...][None, :], be2_ref[...][None, :])
        h = jnp.maximum(jnp.dot(y, wo1_ref[...],
                                preferred_element_type=jnp.float32)
                        + bo1_ref[...][None, :], 0.0)
        out_ref[...] = jnp.dot(h, wo2_ref[...],
                               preferred_element_type=jnp.float32) + bo2_ref[...][None, :]

    out_dim = Wo2.shape[1]
    return pl.pallas_call(
        body,
        out_shape=jax.ShapeDtypeStruct((n, out_dim), jnp.float32),
        compiler_params=pltpu.CompilerParams(
            vmem_limit_bytes=120 * 1024 * 1024),
    )(a, xs2, dinv, b2, g2, be2, Wo1, bo1, Wo2, bo2)


def kernel(X, edge_index, W_in, b_in, W1, b1, g1, be1, W2, b2, g2, be2,
           Wo1, bo1, Wo2, bo2):
    n = X.shape[0]
    e = edge_index.shape[1]
    d = W1.shape[1]

    n_pad = ((n + NS * CH - 1) // (NS * CH)) * NS * CH
    e_pad = ((e + NW * CH - 1) // (NW * CH)) * NW * CH
    nch_deg = e_pad // (NW * CH)
    nch_mp = e_pad // (NS * CH)
    pad = e_pad - e

    src_p = jnp.concatenate([edge_index[0], jnp.zeros((pad,), jnp.int32)])
    dst_p = jnp.concatenate([edge_index[1], jnp.full((pad,), n, jnp.int32)])
    dst3_deg = dst_p.reshape(NW, nch_deg, CH)
    src3 = src_p.reshape(NS, nch_mp, CH)
    dst3 = dst_p.reshape(NS, nch_mp, CH)

    xl = X[:, :, -1]

    deg = _sc_degree(dst3_deg, n_pad, nch_deg)
    d3 = deg.reshape(NC, n_pad, 1)
    dinv, xs1 = _tc_pre(xl, d3, W_in, b_in, W1, n)
    a1 = _sc_msgpass(xs1, src3, dst3, n_pad, nch_mp, n, d)
    xs2 = _tc_mid(a1, xs1, dinv, b1, g1, be1, W2, n)
    a2 = _sc_msgpass(xs2, src3, dst3, n_pad, nch_mp, n, d)
    return _tc_post(a2, xs2, dinv, b2, g2, be2, Wo1, bo1, Wo2, bo2, n)


# speedup vs baseline: 25.9612x; 1.0199x over previous
import functools

import jax
import jax.numpy as jnp
from jax import lax
from jax.experimental import pallas as pl
from jax.experimental.pallas import tpu as pltpu
from jax.experimental.pallas import tpu_sc as plsc

NC, NS = 2, 16
NW = NC * NS
CH = 128
NBUF = 4
NQ = 2 * NC


def _sc_degree(dst3, n_pad, nch):
    mesh = plsc.VectorSubcoreMesh(core_axis_name="c", subcore_axis_name="s")

    @functools.partial(
        pl.kernel,
        out_type=jax.ShapeDtypeStruct((NC, n_pad), jnp.float32),
        mesh=mesh,
        scratch_types=[
            pltpu.VMEM((nch, CH), jnp.int32),
            pltpu.VMEM((CH,), jnp.float32),
            pltpu.VMEM((n_pad // NS,), jnp.float32),
            pltpu.VMEM_SHARED((n_pad,), jnp.float32),
            pltpu.SemaphoreType.DMA,
        ],
    )
    def deg_kernel(dst_hbm, out_hbm, didx, ones_v, zslab, acc_sh, sem):
        c = lax.axis_index("c")
        s = lax.axis_index("s")
        wid = s * NC + c
        pltpu.sync_copy(dst_hbm.at[wid], didx)

        def init(i, _):
            ones_v[pl.ds(i * 16, 16)] = jnp.ones((16,), jnp.float32)
            return 0
        lax.fori_loop(0, CH // 16, init, 0)

        slab = n_pad // NS

        def zinit(i, _):
            zslab[pl.ds(i * 16, 16)] = jnp.zeros((16,), jnp.float32)
            return 0
        lax.fori_loop(0, slab // 16, zinit, 0)
        pltpu.sync_copy(zslab, acc_sh.at[pl.ds(s * slab, slab)])
        plsc.subcore_barrier()

        cps = [pltpu.async_copy(ones_v, acc_sh.at[didx.at[j]], sem, add=True)
               for j in range(nch)]
        for cp in cps:
            cp.wait()
        plsc.subcore_barrier()
        pltpu.sync_copy(acc_sh.at[pl.ds(s * slab, slab)],
                        out_hbm.at[c].at[pl.ds(s * slab, slab)])

    return deg_kernel(dst3)


def _sc_msgpass(xs, src3, dst3, n_pad, nch, n, d):
    mesh = plsc.VectorSubcoreMesh(core_axis_name="c", subcore_axis_name="s")
    dq = d // NQ
    rps = n_pad // (NS * CH)
    rows_st = n // NS

    @functools.partial(
        pl.kernel,
        out_type=jax.ShapeDtypeStruct((n_pad, d), jnp.float32),
        mesh=mesh,
        scratch_types=[
            pltpu.VMEM((nch, CH), jnp.int32),
            pltpu.VMEM((nch, CH), jnp.int32),
            pltpu.VMEM((NBUF, CH, dq), jnp.float32),
            pltpu.VMEM_SHARED((2, n, dq), jnp.float32),
            pltpu.VMEM_SHARED((n_pad, dq), jnp.float32),
        ] + [pltpu.SemaphoreType.DMA] * (2 * NBUF),
        compiler_params=pltpu.CompilerParams(use_tc_tiling_on_sc=False),
    )
    def mp_kernel(xs_hbm, src_hbm, dst_hbm, out_hbm,
                  sidx, didx, ring, tbl_sh, acc_sh, *sems):
        gsem = sems[:NBUF]
        ssem = sems[NBUF:]
        c = lax.axis_index("c")
        s = lax.axis_index("s")

        strows = pl.ds(s * rows_st, rows_st)
        stg = [pltpu.async_copy(
                   xs_hbm.at[strows, pl.ds((2 * c + i) * dq, dq)],
                   tbl_sh.at[i].at[strows], gsem[i])
               for i in range(2)]
        pltpu.sync_copy(src_hbm.at[s], sidx)
        pltpu.sync_copy(dst_hbm.at[s], didx)

        def zr(r, _):
            for cc in range(dq // 16):
                ring[0, r, pl.ds(cc * 16, 16)] = jnp.zeros((16,), jnp.float32)
            return 0

        for i in range(2):
            qcol = (2 * c + i) * dq
            lax.fori_loop(0, CH, zr, 0)
            for k in range(rps):
                pltpu.sync_copy(ring.at[0], acc_sh.at[pl.ds((s * rps + k) * CH, CH)])
            if i == 0:
                stg[0].wait()
                stg[1].wait()
            plsc.subcore_barrier()

            tbl = tbl_sh.at[i]
            gcp = [None] * NBUF
            scp = [None] * NBUF
            for k in range(min(NBUF, nch)):
                gcp[k] = pltpu.async_copy(tbl.at[sidx.at[k]], ring.at[k], gsem[k])
            for j in range(nch):
                b = j % NBUF
                gcp[b].wait()
                scp[b] = pltpu.async_copy(ring.at[b], acc_sh.at[didx.at[j]],
                                          ssem[b], add=True)
                jl = j - (NBUF - 2)
                nj = jl + NBUF
                if jl >= 0 and nj < nch:
                    bl = jl % NBUF
                    scp[bl].wait()
                    gcp[bl] = pltpu.async_copy(tbl.at[sidx.at[nj]], ring.at[bl],
                                               gsem[bl])
            for j in range(max(nch - NBUF, 0), nch):
                scp[j % NBUF].wait()
            plsc.subcore_barrier()
            for k in range(rps):
                off = (s * rps + k) * CH
                pltpu.sync_copy(acc_sh.at[pl.ds(off, CH)],
                                out_hbm.at[pl.ds(off, CH), pl.ds(qcol, dq)])

    return mp_kernel(xs, src3, dst3)


def _tc_pre(xl, d3, W_in, b_in, W1, n):
    h1 = W1.shape[1]

    def body(xl_ref, d_ref, wi_ref, bi_ref, w1_ref, dinv_ref, xs1_ref):
        deg = d_ref[0, :n, :] + d_ref[1, :n, :] + 1.0
        dinv = lax.rsqrt(deg)
        h0 = jnp.dot(xl_ref[...], wi_ref[...],
                     preferred_element_type=jnp.float32) + bi_ref[...][None, :]
        xs1 = jnp.dot(h0, w1_ref[...],
                      preferred_element_type=jnp.float32) * dinv
        dinv_ref[...] = dinv
        xs1_ref[...] = xs1

    return pl.pallas_call(
        body,
        out_shape=(jax.ShapeDtypeStruct((n, 1), jnp.float32),
                   jax.ShapeDtypeStruct((n, h1), jnp.float32)),
        compiler_params=pltpu.CompilerParams(
            vmem_limit_bytes=120 * 1024 * 1024),
    )(xl, d3, W_in, b_in, W1)


def _bn(y, g, b):
    mu = jnp.mean(y, axis=0)
    var = jnp.mean((y - mu) ** 2, axis=0)
    return (y - mu) * lax.rsqrt(var + 1e-5) * g + b


def _tc_mid(a, xs1, dinv, b1, g1, be1, W2, n):
    h2 = W2.shape[1]

    def body(a_ref, xs1_ref, dinv_ref, b1_ref, g1_ref, be1_ref, w2_ref, xs2_ref):
        dinv = dinv_ref[...]
        agg = a_ref[:n, :] + xs1_ref[...]
        y = agg * dinv + b1_ref[...][None, :]
        y = _bn(y, g1_ref[...][None, :], be1_ref[...][None, :])
        y = jnp.maximum(y, 0.0)
        xs2_ref[...] = jnp.dot(y, w2_ref[...],
                               preferred_element_type=jnp.float32) * dinv

    return pl.pallas_call(
        body,
        out_shape=jax.ShapeDtypeStruct((n, h2), jnp.float32),
        compiler_params=pltpu.CompilerParams(
            vmem_limit_bytes=120 * 1024 * 1024),
    )(a, xs1, dinv, b1, g1, be1, W2)


def _tc_post(a, xs2, dinv, b2, g2, be2, Wo1, bo1, Wo2, bo2, n):
    def body(a_ref, xs2_ref, dinv_ref, b2_ref, g2_ref, be2_ref,
             wo1_ref, bo1_ref, wo2_ref, bo2_ref, out_ref):
        dinv = dinv_ref[...]
        agg = a_ref[:n, :] + xs2_ref[...]
        y = agg * dinv + b2_ref[...][None, :]
        y = _bn(y, g2_ref[...][None, :], be2_ref[...][None, :])
        h = jnp.maximum(jnp.dot(y, wo1_ref[...],
                                preferred_element_type=jnp.float32)
                        + bo1_ref[...][None, :], 0.0)
        out_ref[...] = jnp.dot(h, wo2_ref[...],
                               preferred_element_type=jnp.float32) + bo2_ref[...][None, :]

    out_dim = Wo2.shape[1]
    return pl.pallas_call(
        body,
        out_shape=jax.ShapeDtypeStruct((n, out_dim), jnp.float32),
        compiler_params=pltpu.CompilerParams(
            vmem_limit_bytes=120 * 1024 * 1024),
    )(a, xs2, dinv, b2, g2, be2, Wo1, bo1, Wo2, bo2)


def kernel(X, edge_index, W_in, b_in, W1, b1, g1, be1, W2, b2, g2, be2,
           Wo1, bo1, Wo2, bo2):
    n = X.shape[0]
    e = edge_index.shape[1]
    d = W1.shape[1]

    n_pad = ((n + NS * CH - 1) // (NS * CH)) * NS * CH
    e_pad = ((e + NW * CH - 1) // (NW * CH)) * NW * CH
    nch_deg = e_pad // (NW * CH)
    nch_mp = e_pad // (NS * CH)
    pad = e_pad - e

    src_p = jnp.concatenate([edge_index[0], jnp.zeros((pad,), jnp.int32)])
    dst_p = jnp.concatenate([edge_index[1], jnp.full((pad,), n, jnp.int32)])
    dst3_deg = dst_p.reshape(NW, nch_deg, CH)
    src3 = src_p.reshape(NS, nch_mp, CH)
    dst3 = dst_p.reshape(NS, nch_mp, CH)

    xl = X[:, :, -1]

    deg = _sc_degree(dst3_deg, n_pad, nch_deg)
    d3 = deg.reshape(NC, n_pad, 1)
    dinv, xs1 = _tc_pre(xl, d3, W_in, b_in, W1, n)
    a1 = _sc_msgpass(xs1, src3, dst3, n_pad, nch_mp, n, d)
    xs2 = _tc_mid(a1, xs1, dinv, b1, g1, be1, W2, n)
    a2 = _sc_msgpass(xs2, src3, dst3, n_pad, nch_mp, n, d)
    return _tc_post(a2, xs2, dinv, b2, g2, be2, Wo1, bo1, Wo2, bo2, n)


# speedup vs baseline: 26.1832x; 1.0086x over previous
import functools

import jax
import jax.numpy as jnp
from jax import lax
from jax.experimental import pallas as pl
from jax.experimental.pallas import tpu as pltpu
from jax.experimental.pallas import tpu_sc as plsc

NC, NS = 2, 16
NW = NC * NS
CH = 128
NBUF = 4
NQ = 2 * NC


def _sc_degree(dst2, n_pad, epb):
    mesh = plsc.VectorSubcoreMesh(core_axis_name="c", subcore_axis_name="s")
    base = epb // NW
    extra = epb % NW
    nrows = base + (1 if extra else 0)

    @functools.partial(
        pl.kernel,
        out_type=jax.ShapeDtypeStruct((NC, n_pad), jnp.float32),
        mesh=mesh,
        scratch_types=[
            pltpu.VMEM((nrows, CH), jnp.int32),
            pltpu.VMEM((CH,), jnp.float32),
            pltpu.VMEM((n_pad // NS,), jnp.float32),
            pltpu.VMEM_SHARED((n_pad,), jnp.float32),
            pltpu.SemaphoreType.DMA,
        ],
        compiler_params=pltpu.CompilerParams(use_tc_tiling_on_sc=False),
    )
    def deg_kernel(dst_hbm, out_hbm, didx, ones_v, zslab, acc_sh, sem):
        c = lax.axis_index("c")
        s = lax.axis_index("s")
        wid = s * NC + c
        start = wid * base + jnp.minimum(wid, extra)
        pltpu.sync_copy(dst_hbm.at[pl.ds(start, base)], didx.at[pl.ds(0, base)])
        if extra:
            @pl.when(wid < extra)
            def _():
                pltpu.sync_copy(dst_hbm.at[pl.ds(start + base, 1)],
                                didx.at[pl.ds(base, 1)])

        def init(i, _):
            ones_v[pl.ds(i * 16, 16)] = jnp.ones((16,), jnp.float32)
            return 0
        lax.fori_loop(0, CH // 16, init, 0)

        slab = n_pad // NS

        def zinit(i, _):
            zslab[pl.ds(i * 16, 16)] = jnp.zeros((16,), jnp.float32)
            return 0
        lax.fori_loop(0, slab // 16, zinit, 0)
        pltpu.sync_copy(zslab, acc_sh.at[pl.ds(s * slab, slab)])
        plsc.subcore_barrier()

        cps = [pltpu.async_copy(ones_v, acc_sh.at[didx.at[j]], sem, add=True)
               for j in range(base)]
        for cp in cps:
            cp.wait()
        if extra:
            @pl.when(wid < extra)
            def _():
                pltpu.sync_copy(ones_v, acc_sh.at[didx.at[base]], add=True)
        plsc.subcore_barrier()
        pltpu.sync_copy(acc_sh.at[pl.ds(s * slab, slab)],
                        out_hbm.at[c].at[pl.ds(s * slab, slab)])

    return deg_kernel(dst2)


def _sc_msgpass(xs, src2, dst2, n_pad, epb, n, d):
    mesh = plsc.VectorSubcoreMesh(core_axis_name="c", subcore_axis_name="s")
    dq = d // NQ
    rps = n_pad // (NS * CH)
    rows_st = n // NS
    base = epb // NS
    extra = epb % NS
    nrows = base + (1 if extra else 0)

    @functools.partial(
        pl.kernel,
        out_type=jax.ShapeDtypeStruct((n_pad, d), jnp.float32),
        mesh=mesh,
        scratch_types=[
            pltpu.VMEM((nrows, CH), jnp.int32),
            pltpu.VMEM((nrows, CH), jnp.int32),
            pltpu.VMEM((NBUF, CH, dq), jnp.float32),
            pltpu.VMEM_SHARED((2, n, dq), jnp.float32),
            pltpu.VMEM_SHARED((n_pad, dq), jnp.float32),
        ] + [pltpu.SemaphoreType.DMA] * (2 * NBUF),
        compiler_params=pltpu.CompilerParams(use_tc_tiling_on_sc=False),
    )
    def mp_kernel(xs_hbm, src_hbm, dst_hbm, out_hbm,
                  sidx, didx, ring, tbl_sh, acc_sh, *sems):
        gsem = sems[:NBUF]
        ssem = sems[NBUF:]
        c = lax.axis_index("c")
        s = lax.axis_index("s")

        strows = pl.ds(s * rows_st, rows_st)
        stg = [pltpu.async_copy(
                   xs_hbm.at[strows, pl.ds((2 * c + i) * dq, dq)],
                   tbl_sh.at[i].at[strows], gsem[i])
               for i in range(2)]
        start = s * base + jnp.minimum(s, extra)
        pltpu.sync_copy(src_hbm.at[pl.ds(start, base)], sidx.at[pl.ds(0, base)])
        pltpu.sync_copy(dst_hbm.at[pl.ds(start, base)], didx.at[pl.ds(0, base)])
        if extra:
            @pl.when(s < extra)
            def _():
                pltpu.sync_copy(src_hbm.at[pl.ds(start + base, 1)],
                                sidx.at[pl.ds(base, 1)])
                pltpu.sync_copy(dst_hbm.at[pl.ds(start + base, 1)],
                                didx.at[pl.ds(base, 1)])

        def zr(r, _):
            for cc in range(dq // 16):
                ring[0, r, pl.ds(cc * 16, 16)] = jnp.zeros((16,), jnp.float32)
            return 0

        for i in range(2):
            qcol = (2 * c + i) * dq
            lax.fori_loop(0, CH, zr, 0)
            for k in range(rps):
                pltpu.sync_copy(ring.at[0], acc_sh.at[pl.ds((s * rps + k) * CH, CH)])
            if i == 0:
                stg[0].wait()
                stg[1].wait()
            plsc.subcore_barrier()

            tbl = tbl_sh.at[i]
            gcp = [None] * NBUF
            scp = [None] * NBUF
            for k in range(min(NBUF, base)):
                gcp[k] = pltpu.async_copy(tbl.at[sidx.at[k]], ring.at[k], gsem[k])
            for j in range(base):
                b = j % NBUF
                gcp[b].wait()
                scp[b] = pltpu.async_copy(ring.at[b], acc_sh.at[didx.at[j]],
                                          ssem[b], add=True)
                jl = j - (NBUF - 2)
                nj = jl + NBUF
                if jl >= 0 and nj < base:
                    bl = jl % NBUF
                    scp[bl].wait()
                    gcp[bl] = pltpu.async_copy(tbl.at[sidx.at[nj]], ring.at[bl],
                                               gsem[bl])
            for j in range(max(base - NBUF, 0), base):
                scp[j % NBUF].wait()
            if extra:
                @pl.when(s < extra)
                def _():
                    pltpu.async_copy(tbl.at[sidx.at[base]], ring.at[0],
                                     gsem[0]).wait()
                    pltpu.sync_copy(ring.at[0], acc_sh.at[didx.at[base]],
                                    add=True)
            plsc.subcore_barrier()
            for k in range(rps):
                off = (s * rps + k) * CH
                pltpu.sync_copy(acc_sh.at[pl.ds(off, CH)],
                                out_hbm.at[pl.ds(off, CH), pl.ds(qcol, dq)])

    return mp_kernel(xs, src2, dst2)


def _tc_pre(xl, d3, W_in, b_in, W1, n):
    h1 = W1.shape[1]

    def body(xl_ref, d_ref, wi_ref, bi_ref, w1_ref, dinv_ref, xs1_ref):
        deg = d_ref[0, :n, :] + d_ref[1, :n, :] + 1.0
        dinv = lax.rsqrt(deg)
        h0 = jnp.dot(xl_ref[...], wi_ref[...],
                     preferred_element_type=jnp.float32) + bi_ref[...][None, :]
        xs1 = jnp.dot(h0, w1_ref[...],
                      preferred_element_type=jnp.float32) * dinv
        dinv_ref[...] = dinv
        xs1_ref[...] = xs1

    return pl.pallas_call(
        body,
        out_shape=(jax.ShapeDtypeStruct((n, 1), jnp.float32),
                   jax.ShapeDtypeStruct((n, h1), jnp.float32)),
        compiler_params=pltpu.CompilerParams(
            vmem_limit_bytes=120 * 1024 * 1024),
    )(xl, d3, W_in, b_in, W1)


def _bn(y, g, b):
    mu = jnp.mean(y, axis=0)
    var = jnp.mean((y - mu) ** 2, axis=0)
    return (y - mu) * lax.rsqrt(var + 1e-5) * g + b


def _tc_mid(a, xs1, dinv, b1, g1, be1, W2, n):
    h2 = W2.shape[1]

    def body(a_ref, xs1_ref, dinv_ref, b1_ref, g1_ref, be1_ref, w2_ref, xs2_ref):
        dinv = dinv_ref[...]
        agg = a_ref[:n, :] + xs1_ref[...]
        y = agg * dinv + b1_ref[...][None, :]
        y = _bn(y, g1_ref[...][None, :], be1_ref[...][None, :])
        y = jnp.maximum(y, 0.0)
        xs2_ref[...] = jnp.dot(y, w2_ref[...],
                               preferred_element_type=jnp.float32) * dinv

    return pl.pallas_call(
        body,
        out_shape=jax.ShapeDtypeStruct((n, h2), jnp.float32),
        compiler_params=pltpu.CompilerParams(
            vmem_limit_bytes=120 * 1024 * 1024),
    )(a, xs1, dinv, b1, g1, be1, W2)


def _tc_post(a, xs2, dinv, b2, g2, be2, Wo1, bo1, Wo2, bo2, n):
    def body(a_ref, xs2_ref, dinv_ref, b2_ref, g2_ref, be2_ref,
             wo1_ref, bo1_ref, wo2_ref, bo2_ref, out_ref):
        dinv = dinv_ref[...]
        agg = a_ref[:n, :] + xs2_ref[...]
        y = agg * dinv + b2_ref[...][None, :]
        y = _bn(y, g2_ref[...][None, :], be2_ref[...][None, :])
        h = jnp.maximum(jnp.dot(y, wo1_ref[...],
                                preferred_element_type=jnp.float32)
                        + bo1_ref[...][None, :], 0.0)
        out_ref[...] = jnp.dot(h, wo2_ref[...],
                               preferred_element_type=jnp.float32) + bo2_ref[...][None, :]

    out_dim = Wo2.shape[1]
    return pl.pallas_call(
        body,
        out_shape=jax.ShapeDtypeStruct((n, out_dim), jnp.float32),
        compiler_params=pltpu.CompilerParams(
            vmem_limit_bytes=120 * 1024 * 1024),
    )(a, xs2, dinv, b2, g2, be2, Wo1, bo1, Wo2, bo2)


def kernel(X, edge_index, W_in, b_in, W1, b1, g1, be1, W2, b2, g2, be2,
           Wo1, bo1, Wo2, bo2):
    n = X.shape[0]
    e = edge_index.shape[1]
    d = W1.shape[1]

    n_pad = ((n + NS * CH - 1) // (NS * CH)) * NS * CH
    e_pad = ((e + CH - 1) // CH) * CH
    if e_pad != e:
        pad = e_pad - e
        src_f = jnp.concatenate([edge_index[0], jnp.zeros((pad,), jnp.int32)])
        dst_f = jnp.concatenate([edge_index[1], jnp.full((pad,), n, jnp.int32)])
        src2 = src_f.reshape(e_pad // CH, CH)
        dst2 = dst_f.reshape(e_pad // CH, CH)
    else:
        e2 = edge_index.reshape(2, e // CH, CH)
        src2, dst2 = e2[0], e2[1]
    epb = e_pad // CH

    xl = X[:, :, -1]

    deg = _sc_degree(dst2, n_pad, epb)
    d3 = deg.reshape(NC, n_pad, 1)
    dinv, xs1 = _tc_pre(xl, d3, W_in, b_in, W1, n)
    a1 = _sc_msgpass(xs1, src2, dst2, n_pad, epb, n, d)
    xs2 = _tc_mid(a1, xs1, dinv, b1, g1, be1, W2, n)
    a2 = _sc_msgpass(xs2, src2, dst2, n_pad, epb, n, d)
    return _tc_post(a2, xs2, dinv, b2, g2, be2, Wo1, bo1, Wo2, bo2, n)
